# Initial kernel scaffold; baseline (speedup 1.0000x reference)
import jax
import jax.numpy as jnp
from jax.experimental import pallas as pl


def kernel(x, edge_index, batch, params):
    raise NotImplementedError("write your pallas kernel here")


# speedup vs baseline: 2.5349x; 2.5349x over previous
import functools

import jax
import jax.numpy as jnp
from jax import lax
from jax.experimental import pallas as pl
from jax.experimental.pallas import tpu as pltpu
from jax.experimental.pallas import tpu_sc as plsc

_BN_S = 1.0000049999875

_NC = 2
_NS = 16
_K = 80


def _proj_body(first, h_ref, p_ref, wcat_ref, bcat_ref, wm2_ref, bm2_ref,
               cm_ref, betam_ref, u_ref, v_ref, upd_ref):
    h = h_ref[...]
    if not first:
        h = h + jnp.sum(p_ref[...], axis=0)
    z = jnp.dot(h, wcat_ref[...], preferred_element_type=jnp.float32) + bcat_ref[...]
    hh = u_ref.shape[1]
    u_ref[...] = z[:, :hh]
    v_ref[...] = z[:, hh:2 * hh]
    t1 = jnp.maximum(z[:, 2 * hh:], 0.0)
    t2 = jnp.maximum(
        jnp.dot(t1, wm2_ref[...], preferred_element_type=jnp.float32) + bm2_ref[...],
        0.0)
    upd_ref[...] = cm_ref[...] * t2 + betam_ref[...]


def _project(h, p_parts, lp, *, blk=1000):
    n, d = h.shape
    hcap = lp["Wm2"].shape[0]
    first = p_parts is None
    wcat = jnp.concatenate(
        [lp["Wu"][:d], lp["Wd"][d:], lp["Wu"][d:], lp["Wd"][:d], lp["Wm1"]], axis=1)
    zb = jnp.zeros_like(lp["bu"])
    bcat = jnp.concatenate([lp["bu"], zb, zb, lp["bd"], lp["bm1"]])[None, :]
    cm = (lp["gm"] / _BN_S)[None, :]
    betam = lp["betam"][None, :]
    grid = (n // blk,)
    cols = wcat.shape[1]
    in_specs = [
        pl.BlockSpec((blk, d), lambda i: (i, 0)),
        pl.BlockSpec((4, blk, hcap), lambda i: (0, i, 0)),
        pl.BlockSpec((d, cols), lambda i: (0, 0)),
        pl.BlockSpec((1, cols), lambda i: (0, 0)),
        pl.BlockSpec((hcap, hcap), lambda i: (0, 0)),
        pl.BlockSpec((1, hcap), lambda i: (0, 0)),
        pl.BlockSpec((1, hcap), lambda i: (0, 0)),
        pl.BlockSpec((1, hcap), lambda i: (0, 0)),
    ]
    out_specs = [
        pl.BlockSpec((blk, 2 * hcap), lambda i: (i, 0)),
        pl.BlockSpec((blk, 2 * hcap), lambda i: (i, 0)),
        pl.BlockSpec((blk, hcap), lambda i: (i, 0)),
    ]
    out_shape = [
        jax.ShapeDtypeStruct((n, 2 * hcap), jnp.float32),
        jax.ShapeDtypeStruct((n, 2 * hcap), jnp.float32),
        jax.ShapeDtypeStruct((n, hcap), jnp.float32),
    ]
    if first:
        p_parts = jnp.zeros((4, n, hcap), jnp.float32)
    return pl.pallas_call(
        functools.partial(_proj_body, first),
        grid=grid, in_specs=in_specs, out_specs=out_specs, out_shape=out_shape,
    )(h, p_parts, wcat, bcat, lp["Wm2"], lp["bm2"][None, :], cm, betam)


def _head_body(nblk, blk, hu_ref, p_ref, batch_ref, w1_ref, b1_ref,
               w2_ref, b2_ref, out_ref, acc_ref):
    i = pl.program_id(0)
    h = hu_ref[...] + jnp.sum(p_ref[...], axis=0)
    bvec = batch_ref[0, 0, :]
    nb = acc_ref.shape[0]
    onehot = (lax.broadcasted_iota(jnp.int32, (nb, blk), 0) == bvec[None, :])
    m = onehot.astype(jnp.float32)
    hext = jnp.concatenate([h, jnp.ones_like(h)], axis=1)
    part = jnp.dot(m, hext, preferred_element_type=jnp.float32)

    @pl.when(i == 0)
    def _init():
        acc_ref[...] = jnp.zeros_like(acc_ref)

    acc_ref[...] += part

    @pl.when(i == nblk - 1)
    def _fin():
        a = acc_ref[...]
        hcap = a.shape[1] // 2
        pooled = a[:, :hcap] / jnp.maximum(a[:, hcap:hcap + 1], 1.0)
        o1 = jnp.maximum(
            jnp.dot(pooled, w1_ref[...], preferred_element_type=jnp.float32)
            + b1_ref[...], 0.0)
        o2 = (jnp.dot(o1, w2_ref[...], preferred_element_type=jnp.float32)
              + b2_ref[...])
        mx = jnp.max(o2, axis=1, keepdims=True)
        lse = jnp.log(jnp.sum(jnp.exp(o2 - mx), axis=1, keepdims=True)) + mx
        out_ref[...] = o2 - lse


def _head(hu, p_parts, batch_r, params, *, nb, blk=1000):
    n, hcap = hu.shape
    c = params["W2"].shape[1]
    cpad = 128
    w2p = jnp.zeros((hcap, cpad), jnp.float32).at[:, :c].set(params["W2"])
    b2p = jnp.full((1, cpad), -1e30, jnp.float32).at[0, :c].set(params["b2"])
    nblk = n // blk
    return pl.pallas_call(
        functools.partial(_head_body, nblk, blk),
        grid=(nblk,),
        in_specs=[
            pl.BlockSpec((blk, hcap), lambda i: (i, 0)),
            pl.BlockSpec((4, blk, hcap), lambda i: (0, i, 0)),
            pl.BlockSpec((1, 1, blk), lambda i: (i, 0, 0)),
            pl.BlockSpec((hcap, hcap), lambda i: (0, 0)),
            pl.BlockSpec((1, hcap), lambda i: (0, 0)),
            pl.BlockSpec((hcap, cpad), lambda i: (0, 0)),
            pl.BlockSpec((1, cpad), lambda i: (0, 0)),
        ],
        out_specs=pl.BlockSpec((nb, cpad), lambda i: (0, 0)),
        out_shape=jax.ShapeDtypeStruct((nb, cpad), jnp.float32),
        scratch_shapes=[pltpu.VMEM((nb, 2 * hcap), jnp.float32)],
    )(hu, p_parts, batch_r, params["W1"], params["b1"][None, :], w2p, b2p)


def _edge_sc_body(n, e, h, u_hbm, v_hbm, dst_hbm, src_hbm, consts_hbm,
                  zeros_hbm, p_hbm, idxd, idxs, bufu, bufv, msgu, msgd,
                  cbuf, sem, accu, accd):
    nsl = h // 16
    epw = e // (_NC * _NS)
    nchunk = epw // _K
    rows = n // _NS
    c = lax.axis_index("c")
    s = lax.axis_index("s")
    wid = s * _NC + c
    pltpu.sync_copy(zeros_hbm.at[pl.ds(s * rows, rows)],
                    accu.at[pl.ds(s * rows, rows)])
    pltpu.sync_copy(zeros_hbm.at[pl.ds(s * rows, rows)],
                    accd.at[pl.ds(s * rows, rows)])
    pltpu.sync_copy(consts_hbm, cbuf)
    plsc.subcore_barrier()
    cu = [cbuf[0, pl.ds(16 * j, 16)] for j in range(nsl)]
    bu = [cbuf[1, pl.ds(16 * j, 16)] for j in range(nsl)]
    cd = [cbuf[2, pl.ds(16 * j, 16)] for j in range(nsl)]
    bd = [cbuf[3, pl.ds(16 * j, 16)] for j in range(nsl)]
    ebase = wid * epw

    def chunk(i, carry):
        base = ebase + i * _K
        pltpu.sync_copy(dst_hbm.at[pl.ds(base, _K)], idxd)
        pltpu.sync_copy(src_hbm.at[pl.ds(base, _K)], idxs)
        pltpu.async_copy(u_hbm.at[idxd], bufu, sem).wait()
        pltpu.async_copy(v_hbm.at[idxs], bufv, sem).wait()

        def edge(ei, cc):
            for j in range(nsl):
                zu = bufu[ei, pl.ds(16 * j, 16)] + bufv[ei, pl.ds(16 * j, 16)]
                msgu[ei, pl.ds(16 * j, 16)] = cu[j] * jnp.maximum(zu, 0.0) + bu[j]
                zd = bufu[ei, pl.ds(h + 16 * j, 16)] + bufv[ei, pl.ds(h + 16 * j, 16)]
                msgd[ei, pl.ds(16 * j, 16)] = cd[j] * jnp.maximum(zd, 0.0) + bd[j]
            return cc

        lax.fori_loop(0, _K, edge, 0)
        pltpu.sync_copy(msgu, accu.at[idxd], add=True)
        pltpu.sync_copy(msgd, accd.at[idxs], add=True)
        return carry

    lax.fori_loop(0, nchunk, chunk, 0)
    plsc.subcore_barrier()
    pltpu.sync_copy(accu.at[pl.ds(s * rows, rows)],
                    p_hbm.at[2 * c, pl.ds(s * rows, rows)])
    pltpu.sync_copy(accd.at[pl.ds(s * rows, rows)],
                    p_hbm.at[2 * c + 1, pl.ds(s * rows, rows)])


def _edge_pass(u, v, dst, src, lp):
    n, hh = u.shape
    h = hh // 2
    e = dst.shape[0]
    consts = jnp.stack([lp["gu"] / _BN_S, lp["betau"],
                        lp["gd"] / _BN_S, lp["betad"]])
    zeros = jnp.zeros((n, h), jnp.float32)
    mesh = plsc.VectorSubcoreMesh(core_axis_name="c", subcore_axis_name="s",
                                  num_cores=_NC, num_subcores=_NS)
    kern = pl.kernel(
        functools.partial(_edge_sc_body, n, e, h),
        out_type=jax.ShapeDtypeStruct((4, n, h), jnp.float32),
        mesh=mesh,
        compiler_params=pltpu.CompilerParams(use_tc_tiling_on_sc=False),
        scratch_types=[
            pltpu.VMEM((_K,), jnp.int32),
            pltpu.VMEM((_K,), jnp.int32),
            pltpu.VMEM((_K, hh), jnp.float32),
            pltpu.VMEM((_K, hh), jnp.float32),
            pltpu.VMEM((_K, h), jnp.float32),
            pltpu.VMEM((_K, h), jnp.float32),
            pltpu.VMEM((4, h), jnp.float32),
            pltpu.SemaphoreType.DMA,
            pltpu.VMEM_SHARED((n, h), jnp.float32),
            pltpu.VMEM_SHARED((n, h), jnp.float32),
        ],
    )
    return kern(u, v, dst, src, consts, zeros)


def kernel(x, edge_index, batch, params):
    n = x.shape[0]
    nb = 64
    blk = 1000
    src = edge_index[0].astype(jnp.int32)
    dst = edge_index[1].astype(jnp.int32)
    batch_r = batch.astype(jnp.int32).reshape(n // blk, 1, blk)

    p_parts = None
    hu = x
    for lp in params["layers"]:
        u, v, upd = _project(hu, p_parts, lp, blk=blk)
        p_parts = _edge_pass(u, v, dst, src, lp)
        hu = upd
    out = _head(hu, p_parts, batch_r, params, nb=nb, blk=blk)
    return out[:, :params["W2"].shape[1]]


# speedup vs baseline: 8.6805x; 3.4244x over previous
import functools

import jax
import jax.numpy as jnp
from jax import lax
from jax.experimental import pallas as pl
from jax.experimental.pallas import tpu as pltpu
from jax.experimental.pallas import tpu_sc as plsc

_BN_S = 1.0000049999875

_NC = 2
_NS = 16
_K = 128
_NPAD = 10240
_TRASH = 10200
_BLK = 640


def _proj_body(first, h_ref, p_ref, wcat_ref, bcat_ref, wm2_ref, bm2_ref,
               cm_ref, betam_ref, tdu_ref, tsu_ref, tsd_ref, tdd_ref, upd_ref):
    h = h_ref[...]
    if not first:
        h = h + jnp.sum(p_ref[...], axis=0)
    z = jnp.dot(h, wcat_ref[...], preferred_element_type=jnp.float32) + bcat_ref[...]
    hc = upd_ref.shape[1]
    tdu_ref[...] = z[:, :hc]
    tsu_ref[...] = z[:, hc:2 * hc]
    tsd_ref[...] = z[:, 2 * hc:3 * hc]
    tdd_ref[...] = z[:, 3 * hc:4 * hc]
    t1 = jnp.maximum(z[:, 4 * hc:], 0.0)
    t2 = jnp.maximum(
        jnp.dot(t1, wm2_ref[...], preferred_element_type=jnp.float32) + bm2_ref[...],
        0.0)
    upd_ref[...] = cm_ref[...] * t2 + betam_ref[...]


def _project(h, p_parts, lp):
    n, d = h.shape
    hc = lp["Wm2"].shape[0]
    first = p_parts is None
    wcat = jnp.concatenate(
        [lp["Wu"][:d], lp["Wu"][d:], lp["Wd"][:d], lp["Wd"][d:], lp["Wm1"]], axis=1)
    zb = jnp.zeros_like(lp["bu"])
    bcat = jnp.concatenate([lp["bu"], zb, lp["bd"], zb, lp["bm1"]])[None, :]
    cm = (lp["gm"] / _BN_S)[None, :]
    betam = lp["betam"][None, :]
    grid = (n // _BLK,)
    cols = wcat.shape[1]
    in_specs = [
        pl.BlockSpec((_BLK, d), lambda i: (i, 0)),
        pl.BlockSpec((2, _BLK, hc), lambda i: (0, i, 0)),
        pl.BlockSpec((d, cols), lambda i: (0, 0)),
        pl.BlockSpec((1, cols), lambda i: (0, 0)),
        pl.BlockSpec((hc, hc), lambda i: (0, 0)),
        pl.BlockSpec((1, hc), lambda i: (0, 0)),
        pl.BlockSpec((1, hc), lambda i: (0, 0)),
        pl.BlockSpec((1, hc), lambda i: (0, 0)),
    ]
    out_specs = [pl.BlockSpec((_BLK, hc), lambda i: (i, 0)) for _ in range(5)]
    out_shape = [jax.ShapeDtypeStruct((n, hc), jnp.float32) for _ in range(5)]
    if first:
        p_parts = jnp.zeros((2, n, hc), jnp.float32)
    return pl.pallas_call(
        functools.partial(_proj_body, first),
        grid=grid, in_specs=in_specs, out_specs=out_specs, out_shape=out_shape,
    )(h, p_parts, wcat, bcat, lp["Wm2"], lp["bm2"][None, :], cm, betam)


def _head_body(nblk, hu_ref, p_ref, batch_ref, w1_ref, b1_ref,
               w2_ref, b2_ref, out_ref, acc_ref):
    i = pl.program_id(0)
    h = hu_ref[...] + jnp.sum(p_ref[...], axis=0)
    bvec = batch_ref[0, 0, :]
    nb = acc_ref.shape[0]
    blk = h.shape[0]
    onehot = (lax.broadcasted_iota(jnp.int32, (nb, blk), 0) == bvec[None, :])
    m = onehot.astype(jnp.float32)
    hext = jnp.concatenate([h, jnp.ones_like(h)], axis=1)
    part = jnp.dot(m, hext, preferred_element_type=jnp.float32)

    @pl.when(i == 0)
    def _init():
        acc_ref[...] = jnp.zeros_like(acc_ref)

    acc_ref[...] += part

    @pl.when(i == nblk - 1)
    def _fin():
        a = acc_ref[...]
        hc = a.shape[1] // 2
        pooled = a[:, :hc] / jnp.maximum(a[:, hc:hc + 1], 1.0)
        o1 = jnp.maximum(
            jnp.dot(pooled, w1_ref[...], preferred_element_type=jnp.float32)
            + b1_ref[...], 0.0)
        o2 = (jnp.dot(o1, w2_ref[...], preferred_element_type=jnp.float32)
              + b2_ref[...])
        mx = jnp.max(o2, axis=1, keepdims=True)
        lse = jnp.log(jnp.sum(jnp.exp(o2 - mx), axis=1, keepdims=True)) + mx
        out_ref[...] = o2 - lse


def _head(hu, p_parts, batch_r, params, *, nb):
    n, hc = hu.shape
    c = params["W2"].shape[1]
    cpad = 128
    w2p = jnp.zeros((hc, cpad), jnp.float32).at[:, :c].set(params["W2"])
    b2p = jnp.full((1, cpad), -1e30, jnp.float32).at[0, :c].set(params["b2"])
    nblk = n // _BLK
    return pl.pallas_call(
        functools.partial(_head_body, nblk),
        grid=(nblk,),
        in_specs=[
            pl.BlockSpec((_BLK, hc), lambda i: (i, 0)),
            pl.BlockSpec((2, _BLK, hc), lambda i: (0, i, 0)),
            pl.BlockSpec((1, 1, _BLK), lambda i: (i, 0, 0)),
            pl.BlockSpec((hc, hc), lambda i: (0, 0)),
            pl.BlockSpec((1, hc), lambda i: (0, 0)),
            pl.BlockSpec((hc, cpad), lambda i: (0, 0)),
            pl.BlockSpec((1, cpad), lambda i: (0, 0)),
        ],
        out_specs=pl.BlockSpec((nb, cpad), lambda i: (0, 0)),
        out_shape=jax.ShapeDtypeStruct((nb, cpad), jnp.float32),
        scratch_shapes=[pltpu.VMEM((nb, 2 * hc), jnp.float32)],
    )(hu, p_parts, batch_r, params["W1"], params["b1"][None, :], w2p, b2p)


def _edge_sc_body(n, h, nchunk, tdu_hbm, tsu_hbm, tsd_hbm, tdd_hbm,
                  dst_hbm, src_hbm, consts_hbm, p_hbm,
                  idxd, idxs, bufa0, bufb0, bufa1, bufb1, msg, cbuf,
                  gsem0, gsem1, acc):
    nsl = h // 16
    rows = n // _NS
    c = lax.axis_index("c")
    s = lax.axis_index("s")

    @plsc.parallel_loop(0, _K, 1, unroll=8)
    def _zero(ei):
        for j in range(nsl):
            msg[ei, pl.ds(16 * j, 16)] = jnp.zeros((16,), jnp.float32)

    for r in range(rows // _K):
        pltpu.sync_copy(msg, acc.at[pl.ds(s * rows + r * _K, _K)])

    pltpu.sync_copy(dst_hbm.at[s], idxd)
    pltpu.sync_copy(src_hbm.at[s], idxs)
    pltpu.sync_copy(consts_hbm, cbuf)
    plsc.subcore_barrier()
    m = nchunk // 2

    def run(ta_hbm, tb_hbm, scat_idx, crow):
        cs = [cbuf[crow, pl.ds(16 * j, 16)] for j in range(nsl)]
        cb = [cbuf[crow + 1, pl.ds(16 * j, 16)] for j in range(nsl)]

        def compute(ba_ref, bb_ref):
            @plsc.parallel_loop(0, _K, 1, unroll=4)
            def _edge(ei):
                for j in range(nsl):
                    z = ba_ref[ei, pl.ds(16 * j, 16)] + bb_ref[ei, pl.ds(16 * j, 16)]
                    msg[ei, pl.ds(16 * j, 16)] = (
                        cs[j] * jnp.maximum(z, 0.0) + cb[j])

        def gather(chunk, ba_ref, bb_ref, sem):
            pltpu.async_copy(ta_hbm.at[idxd.at[chunk]], ba_ref, sem)
            pltpu.async_copy(tb_hbm.at[idxs.at[chunk]], bb_ref, sem)

        def gwait(chunk, ba_ref, bb_ref, sem):
            pltpu.make_async_copy(ta_hbm.at[idxd.at[chunk]], ba_ref, sem).wait()
            pltpu.make_async_copy(tb_hbm.at[idxs.at[chunk]], bb_ref, sem).wait()

        def scatter(chunk):
            pltpu.sync_copy(msg, acc.at[scat_idx.at[chunk]], add=True)

        gather(0, bufa0, bufb0, gsem0)
        gather(1, bufa1, bufb1, gsem1)

        def dbl(i2, carry):
            a = 2 * i2
            gwait(a, bufa0, bufb0, gsem0)
            compute(bufa0, bufb0)

            @pl.when(i2 < m - 1)
            def _pf0():
                gather(a + 2, bufa0, bufb0, gsem0)

            scatter(a)
            gwait(a + 1, bufa1, bufb1, gsem1)
            compute(bufa1, bufb1)

            @pl.when(i2 < m - 1)
            def _pf1():
                gather(a + 3, bufa1, bufb1, gsem1)

            scatter(a + 1)
            return carry

        lax.fori_loop(0, m, dbl, 0)

    @pl.when(c == 0)
    def _up():
        run(tdu_hbm, tsu_hbm, idxd, 0)

    @pl.when(c == 1)
    def _dn():
        run(tdd_hbm, tsd_hbm, idxs, 2)

    plsc.subcore_barrier()
    pltpu.sync_copy(acc.at[pl.ds(s * rows, rows)],
                    p_hbm.at[c, pl.ds(s * rows, rows)])


def _edge_pass(tdu, tsu, tsd, tdd, dst3, src3, lp):
    n, h = tdu.shape
    nchunk = dst3.shape[1]
    consts = jnp.stack([lp["gu"] / _BN_S, lp["betau"],
                        lp["gd"] / _BN_S, lp["betad"]])
    mesh = plsc.VectorSubcoreMesh(core_axis_name="c", subcore_axis_name="s",
                                  num_cores=_NC, num_subcores=_NS)
    kern = pl.kernel(
        functools.partial(_edge_sc_body, n, h, nchunk),
        out_type=jax.ShapeDtypeStruct((2, n, h), jnp.float32),
        mesh=mesh,
        compiler_params=pltpu.CompilerParams(use_tc_tiling_on_sc=False),
        scratch_types=[
            pltpu.VMEM((nchunk, _K), jnp.int32),
            pltpu.VMEM((nchunk, _K), jnp.int32),
            pltpu.VMEM((_K, h), jnp.float32),
            pltpu.VMEM((_K, h), jnp.float32),
            pltpu.VMEM((_K, h), jnp.float32),
            pltpu.VMEM((_K, h), jnp.float32),
            pltpu.VMEM((_K, h), jnp.float32),
            pltpu.VMEM((4, h), jnp.float32),
            pltpu.SemaphoreType.DMA,
            pltpu.SemaphoreType.DMA,
            pltpu.VMEM_SHARED((n, h), jnp.float32),
        ],
    )
    return kern(tdu, tsu, tsd, tdd, dst3, src3, consts)


def _pad_edges(idx, e):
    ept = e // _NS
    nchunk = -(-ept // _K)
    if nchunk % 2:
        nchunk += 1
    per = idx.reshape(_NS, ept)
    pad = jnp.full((_NS, nchunk * _K - ept), _TRASH, jnp.int32)
    return jnp.concatenate([per, pad], axis=1).reshape(_NS, nchunk, _K)


def kernel(x, edge_index, batch, params):
    n = x.shape[0]
    nb = 64
    src3 = _pad_edges(edge_index[0].astype(jnp.int32), edge_index.shape[1])
    dst3 = _pad_edges(edge_index[1].astype(jnp.int32), edge_index.shape[1])
    xp = jnp.pad(x, ((0, _NPAD - n), (0, 0)))
    batch_p = jnp.pad(batch.astype(jnp.int32), (0, _NPAD - n),
                      constant_values=nb)
    batch_r = batch_p.reshape(_NPAD // _BLK, 1, _BLK)

    p_parts = None
    hu = xp
    for lp in params["layers"]:
        tdu, tsu, tsd, tdd, upd = _project(hu, p_parts, lp)
        p_parts = _edge_pass(tdu, tsu, tsd, tdd, dst3, src3, lp)
        hu = upd
    out = _head(hu, p_parts, batch_r, params, nb=nb)
    return out[:, :params["W2"].shape[1]]
